# baseline (speedup 1.0000x reference)
import functools

import jax
import jax.numpy as jnp
from jax import lax
from jax.experimental import pallas as pl
from jax.experimental.pallas import tpu as pltpu
from jax.experimental.pallas import tpu_sc as plsc

DIM = 64
NC = 2
NS = 16
NW = NC * NS
LANES = 16
CHUNK = 128


def _rsqrt_newton(x):
    xi = lax.bitcast_convert_type(x, jnp.int32)
    yi = jnp.int32(0x5F3759DF) - (xi >> 1)
    y = lax.bitcast_convert_type(yi, jnp.float32)
    for _ in range(3):
        y = y * (1.5 - 0.5 * x * y * y)
    return y


@functools.partial(jax.jit, static_argnames=("total",))
def _transh_sc(h_idx, r_idx, t_idx, entity_emb, relation_emb, total):
    S = total // NW
    n_chunks = S // CHUNK

    mesh = plsc.VectorSubcoreMesh(core_axis_name="c", subcore_axis_name="s")

    @functools.partial(
        pl.kernel,
        mesh=mesh,
        out_type=jax.ShapeDtypeStruct((total,), jnp.float32),
        compiler_params=pltpu.CompilerParams(
            needs_layout_passes=False, use_tc_tiling_on_sc=False),
        scratch_types=[
            pltpu.VMEM((S,), jnp.int32),
            pltpu.VMEM((S,), jnp.int32),
            pltpu.VMEM((S,), jnp.int32),
            pltpu.VMEM((CHUNK, DIM), jnp.float32),
            pltpu.VMEM((CHUNK, DIM), jnp.float32),
            pltpu.VMEM((CHUNK, 2 * DIM), jnp.float32),
            pltpu.VMEM((S,), jnp.float32),
            pltpu.SemaphoreType.DMA,
            pltpu.SemaphoreType.DMA,
            pltpu.SemaphoreType.DMA,
        ],
    )
    def k(h_hbm, r_hbm, t_hbm, ent_hbm, rel_hbm, out_hbm,
          hidx, ridx, tidx, hbuf, tbuf, rbuf, outv, sem_h, sem_t, sem_r):
        wid = lax.axis_index("s") * NC + lax.axis_index("c")
        base = pl.multiple_of(wid * S, CHUNK)
        pltpu.sync_copy(h_hbm.at[pl.ds(base, S)], hidx)
        pltpu.sync_copy(r_hbm.at[pl.ds(base, S)], ridx)
        pltpu.sync_copy(t_hbm.at[pl.ds(base, S)], tidx)

        for ck in range(n_chunks):
            cbase = ck * CHUNK
            cp_h = pltpu.async_copy(ent_hbm.at[hidx.at[pl.ds(cbase, CHUNK)]],
                                    hbuf, sem_h)
            cp_t = pltpu.async_copy(ent_hbm.at[tidx.at[pl.ds(cbase, CHUNK)]],
                                    tbuf, sem_t)
            cp_r = pltpu.async_copy(rel_hbm.at[ridx.at[pl.ds(cbase, CHUNK)]],
                                    rbuf, sem_r)
            cp_h.wait()
            cp_t.wait()
            cp_r.wait()

            def group_body(g, _, cbase=cbase):
                rows = g * LANES + lax.iota(jnp.int32, LANES)

                def dim_body(d, acc):
                    dd = jnp.full((LANES,), d, jnp.int32)
                    hv = plsc.load_gather(hbuf, [rows, dd])
                    tv = plsc.load_gather(tbuf, [rows, dd])
                    r0 = plsc.load_gather(rbuf, [rows, dd])
                    r1 = plsc.load_gather(rbuf, [rows, dd + DIM])
                    s = hv - tv + r0 + r1
                    return acc + s * s

                acc = lax.fori_loop(0, DIM, dim_body,
                                    jnp.zeros((LANES,), jnp.float32))
                dist = acc * _rsqrt_newton(acc)
                dist = jnp.where(acc > 0.0, dist, 0.0)
                outv[pl.ds(cbase + g * LANES, LANES)] = dist
                return 0

            lax.fori_loop(0, CHUNK // LANES, group_body, 0)

        pltpu.sync_copy(outv, out_hbm.at[pl.ds(base, S)])

    return k(h_idx, r_idx, t_idx, entity_emb, relation_emb)


def kernel(positive_sample, negative_sample, entity_emb, relation_emb):
    b = positive_sample.shape[1]
    h_idx = jnp.concatenate([positive_sample[0], negative_sample[0]])
    r_idx = jnp.concatenate([positive_sample[1], negative_sample[1]])
    t_idx = jnp.concatenate([positive_sample[2], negative_sample[2]])
    dist = _transh_sc(h_idx, r_idx, t_idx, entity_emb, relation_emb, 2 * b)
    return (dist[:b], dist[b:])


# speedup vs baseline: 1.2038x; 1.2038x over previous
import functools

import jax
import jax.numpy as jnp
from jax import lax
from jax.experimental import pallas as pl
from jax.experimental.pallas import tpu as pltpu
from jax.experimental.pallas import tpu_sc as plsc

DIM = 64
NC = 2
NS = 16
NW = NC * NS
LANES = 16
CHUNK = 128
NBUF = 2


def _rsqrt_newton(x):
    xi = lax.bitcast_convert_type(x, jnp.int32)
    yi = jnp.int32(0x5F3759DF) - (xi >> 1)
    y = lax.bitcast_convert_type(yi, jnp.float32)
    for _ in range(3):
        y = y * (1.5 - 0.5 * x * y * y)
    return y


@functools.partial(jax.jit, static_argnames=("total",))
def _transh_sc(h_idx, r_idx, t_idx, entity_emb, relation_emb, total):
    S = total // NW
    n_chunks = S // CHUNK

    mesh = plsc.VectorSubcoreMesh(core_axis_name="c", subcore_axis_name="s")

    @functools.partial(
        pl.kernel,
        mesh=mesh,
        out_type=jax.ShapeDtypeStruct((total,), jnp.float32),
        compiler_params=pltpu.CompilerParams(
            needs_layout_passes=False,
            use_tc_tiling_on_sc=False,
            disable_bounds_checks=True,
        ),
        scratch_types=[
            pltpu.VMEM((S,), jnp.int32),
            pltpu.VMEM((S,), jnp.int32),
            pltpu.VMEM((S,), jnp.int32),
            pltpu.VMEM((NBUF, CHUNK, DIM), jnp.float32),
            pltpu.VMEM((NBUF, CHUNK, DIM), jnp.float32),
            pltpu.VMEM((NBUF, CHUNK, 2 * DIM), jnp.float32),
            pltpu.VMEM((CHUNK * LANES // 8,), jnp.float32),
            pltpu.VMEM((S,), jnp.float32),
            pltpu.SemaphoreType.DMA,
            pltpu.SemaphoreType.DMA,
            pltpu.SemaphoreType.DMA,
            pltpu.SemaphoreType.DMA,
            pltpu.SemaphoreType.DMA,
            pltpu.SemaphoreType.DMA,
        ],
    )
    def k(h_hbm, r_hbm, t_hbm, ent_hbm, rel_hbm, out_hbm,
          hidx, ridx, tidx, hbuf, tbuf, rbuf, trbuf, outv, *sems):
        wid = lax.axis_index("s") * NC + lax.axis_index("c")
        base = pl.multiple_of(wid * S, CHUNK)
        pltpu.sync_copy(h_hbm.at[pl.ds(base, S)], hidx)
        pltpu.sync_copy(r_hbm.at[pl.ds(base, S)], ridx)
        pltpu.sync_copy(t_hbm.at[pl.ds(base, S)], tidx)

        def start(ck):
            bf = ck % NBUF
            cbase = ck * CHUNK
            return (
                pltpu.async_copy(ent_hbm.at[hidx.at[pl.ds(cbase, CHUNK)]],
                                 hbuf.at[bf], sems[3 * bf]),
                pltpu.async_copy(ent_hbm.at[tidx.at[pl.ds(cbase, CHUNK)]],
                                 tbuf.at[bf], sems[3 * bf + 1]),
                pltpu.async_copy(rel_hbm.at[ridx.at[pl.ds(cbase, CHUNK)]],
                                 rbuf.at[bf], sems[3 * bf + 2]),
            )

        lane16 = lax.iota(jnp.int32, LANES) * LANES
        inflight = start(0)

        for ck in range(n_chunks):
            bf = ck % NBUF
            for cp in inflight:
                cp.wait()
            if ck + 1 < n_chunks:
                inflight = start(ck + 1)
            hb, tb, rb = hbuf.at[bf], tbuf.at[bf], rbuf.at[bf]

            def group_body(g, _, hb=hb, tb=tb, rb=rb, cbase=ck * CHUNK):
                gbase = g * LANES

                def sample_body(i, _):
                    s = gbase + i
                    acc = None
                    for c in range(DIM // LANES):
                        lo = c * LANES
                        hv = hb[s, pl.ds(lo, LANES)]
                        tv = tb[s, pl.ds(lo, LANES)]
                        r0 = rb[s, pl.ds(lo, LANES)]
                        r1 = rb[s, pl.ds(DIM + lo, LANES)]
                        d = hv - tv + r0 + r1
                        sq = d * d
                        acc = sq if acc is None else acc + sq
                    trbuf[pl.ds(i * LANES, LANES)] = acc
                    return 0

                lax.fori_loop(0, LANES, sample_body, 0)

                acc = None
                for l in range(LANES):
                    v = plsc.load_gather(trbuf, [lane16 + l])
                    acc = v if acc is None else acc + v
                dist = acc * _rsqrt_newton(acc)
                dist = jnp.where(acc > 0.0, dist, 0.0)
                outv[pl.ds(cbase + gbase, LANES)] = dist
                return 0

            lax.fori_loop(0, CHUNK // LANES, group_body, 0)

        pltpu.sync_copy(outv, out_hbm.at[pl.ds(base, S)])

    return k(h_idx, r_idx, t_idx, entity_emb, relation_emb)


def kernel(positive_sample, negative_sample, entity_emb, relation_emb):
    b = positive_sample.shape[1]
    h_idx = jnp.concatenate([positive_sample[0], negative_sample[0]])
    r_idx = jnp.concatenate([positive_sample[1], negative_sample[1]])
    t_idx = jnp.concatenate([positive_sample[2], negative_sample[2]])
    dist = _transh_sc(h_idx, r_idx, t_idx, entity_emb, relation_emb, 2 * b)
    return (dist[:b], dist[b:])
